# baseline (speedup 1.0000x reference)
import functools
import jax
import jax.numpy as jnp
from jax import lax
from jax.experimental import pallas as pl
from jax.experimental.pallas import tpu as pltpu
from jax.experimental.pallas import tpu_sc as plsc

B = 16384
HIDDEN = 256
NUM_SE = 963
BLK = 4096
NB = B // BLK

NW = 32
B_PER_W = B // NW
CHUNK = 128


def _sc_gather_fn():
    mesh = plsc.VectorSubcoreMesh(core_axis_name="c", subcore_axis_name="s")

    @functools.partial(
        pl.kernel, mesh=mesh,
        out_type=jax.ShapeDtypeStruct((B, HIDDEN), jnp.float32),
        scratch_types=[
            pltpu.VMEM((CHUNK,), jnp.int32),
            pltpu.VMEM((CHUNK, HIDDEN), jnp.float32),
            pltpu.SemaphoreType.DMA,
        ],
    )
    def gk(d_hbm, idx_hbm, out_hbm, idx_v, rows_v, sem):
        wid = lax.axis_index("s") * 2 + lax.axis_index("c")
        base = wid * B_PER_W
        for c in range(B_PER_W // CHUNK):
            o = base + c * CHUNK
            pltpu.sync_copy(idx_hbm.at[pl.ds(o, CHUNK)], idx_v)
            pltpu.async_copy(d_hbm.at[idx_v], rows_v, sem).wait()
            pltpu.sync_copy(rows_v, out_hbm.at[pl.ds(o, CHUNK)])

    return gk


def _tc_body(zi_ref, zj_ref, r_ref, g_ref, out_ref):
    g = g_ref[...]
    d2 = g * g
    rz = jax.lax.dot_general(
        zj_ref[...], r_ref[...],
        dimension_numbers=(((1,), (1,)), ((), ())),
        preferred_element_type=jnp.float32)
    prod = zi_ref[...] * rz * d2
    ones = jnp.ones((8, HIDDEN), dtype=jnp.float32)
    s = jax.lax.dot_general(
        ones, prod,
        dimension_numbers=(((1,), (1,)), ((), ())),
        preferred_element_type=jnp.float32)
    out_ref[0, 0, :] = jax.nn.sigmoid(s[0, :])


def kernel(z_i, z_j, R, D, se_indices):
    se32 = se_indices.astype(jnp.int32)
    rows = _sc_gather_fn()(D, se32)
    out = pl.pallas_call(
        _tc_body,
        grid=(NB,),
        in_specs=[
            pl.BlockSpec((BLK, HIDDEN), lambda i: (i, 0)),
            pl.BlockSpec((BLK, HIDDEN), lambda i: (i, 0)),
            pl.BlockSpec((HIDDEN, HIDDEN), lambda i: (0, 0)),
            pl.BlockSpec((BLK, HIDDEN), lambda i: (i, 0)),
        ],
        out_specs=pl.BlockSpec((1, 1, BLK), lambda i: (i, 0, 0)),
        out_shape=jax.ShapeDtypeStruct((NB, 1, BLK), jnp.float32),
    )(z_i, z_j, R, rows)
    return out.reshape(B)


# speedup vs baseline: 1.0326x; 1.0326x over previous
import functools
import jax
import jax.numpy as jnp
from jax import lax
from jax.experimental import pallas as pl
from jax.experimental.pallas import tpu as pltpu
from jax.experimental.pallas import tpu_sc as plsc

B = 16384
HIDDEN = 256
NUM_SE = 963
BLK = 4096
NB = B // BLK

NW = 32
B_PER_W = B // NW
CHUNK = 128


def _sc_gather_fn():
    mesh = plsc.VectorSubcoreMesh(core_axis_name="c", subcore_axis_name="s")

    @functools.partial(
        pl.kernel, mesh=mesh,
        out_type=jax.ShapeDtypeStruct((B, HIDDEN), jnp.float32),
        scratch_types=[
            pltpu.VMEM((CHUNK,), jnp.int32),
            pltpu.VMEM((CHUNK,), jnp.int32),
            pltpu.VMEM((CHUNK, HIDDEN), jnp.float32),
            pltpu.VMEM((CHUNK, HIDDEN), jnp.float32),
            pltpu.SemaphoreType.DMA,
            pltpu.SemaphoreType.DMA,
            pltpu.SemaphoreType.DMA,
            pltpu.SemaphoreType.DMA,
        ],
    )
    def gk(d_hbm, idx_hbm, out_hbm, idx0, idx1, rows0, rows1,
           gs0, gs1, os0, os1):
        wid = lax.axis_index("s") * 2 + lax.axis_index("c")
        base = wid * B_PER_W
        n = B_PER_W // CHUNK
        bufs = [(idx0, rows0, gs0, os0), (idx1, rows1, gs1, os1)]
        dmas = [None, None]
        outs = [None, None]
        pltpu.sync_copy(idx_hbm.at[pl.ds(base, CHUNK)], idx0)
        dmas[0] = pltpu.async_copy(d_hbm.at[idx0], rows0, gs0)
        for c in range(n):
            ii, rr, gs, os = bufs[c % 2]
            if c + 1 < n:
                i2, r2, g2, _ = bufs[(c + 1) % 2]
                if outs[(c + 1) % 2] is not None:
                    outs[(c + 1) % 2].wait()
                pltpu.sync_copy(
                    idx_hbm.at[pl.ds(base + (c + 1) * CHUNK, CHUNK)], i2)
                dmas[(c + 1) % 2] = pltpu.async_copy(d_hbm.at[i2], r2, g2)
            dmas[c % 2].wait()
            outs[c % 2] = pltpu.async_copy(
                rr, out_hbm.at[pl.ds(base + c * CHUNK, CHUNK)], os)
        outs[(n - 1) % 2].wait()
        outs[(n - 2) % 2].wait()

    return gk


def _tc_body(zi_ref, zj_ref, r_ref, g_ref, out_ref):
    g = g_ref[...]
    d2 = g * g
    rz = jax.lax.dot_general(
        zj_ref[...], r_ref[...],
        dimension_numbers=(((1,), (1,)), ((), ())),
        preferred_element_type=jnp.float32)
    prod = zi_ref[...] * rz * d2
    ones = jnp.ones((8, HIDDEN), dtype=jnp.float32)
    s = jax.lax.dot_general(
        ones, prod,
        dimension_numbers=(((1,), (1,)), ((), ())),
        preferred_element_type=jnp.float32)
    out_ref[0, 0, :] = jax.nn.sigmoid(s[0, :])


def kernel(z_i, z_j, R, D, se_indices):
    se32 = se_indices.astype(jnp.int32)
    rows = _sc_gather_fn()(D, se32)
    out = pl.pallas_call(
        _tc_body,
        grid=(NB,),
        in_specs=[
            pl.BlockSpec((BLK, HIDDEN), lambda i: (i, 0)),
            pl.BlockSpec((BLK, HIDDEN), lambda i: (i, 0)),
            pl.BlockSpec((HIDDEN, HIDDEN), lambda i: (0, 0)),
            pl.BlockSpec((BLK, HIDDEN), lambda i: (i, 0)),
        ],
        out_specs=pl.BlockSpec((1, 1, BLK), lambda i: (i, 0, 0)),
        out_shape=jax.ShapeDtypeStruct((NB, 1, BLK), jnp.float32),
    )(z_i, z_j, R, rows)
    return out.reshape(B)


# speedup vs baseline: 2.6661x; 2.5818x over previous
import jax
import jax.numpy as jnp
from jax.experimental import pallas as pl
from jax.experimental.pallas import tpu as pltpu

B = 16384
HIDDEN = 256
NUM_SE = 963
BLK = 4096
NB = B // BLK


def _body(se_ref, zi_ref, zj_ref, r_ref, d_ref, out_ref):
    idx = se_ref[0, 0, :]
    onehot = (idx[:, None] == jax.lax.broadcasted_iota(
        jnp.int32, (BLK, NUM_SE), 1)).astype(jnp.bfloat16)
    d = d_ref[...]
    d2_tab = (d * d).astype(jnp.bfloat16)
    d2 = jax.lax.dot_general(
        onehot, d2_tab,
        dimension_numbers=(((1,), (0,)), ((), ())),
        preferred_element_type=jnp.float32)
    rz = jax.lax.dot_general(
        zj_ref[...], r_ref[...],
        dimension_numbers=(((1,), (1,)), ((), ())),
        preferred_element_type=jnp.float32)
    prod = zi_ref[...] * rz * d2
    ones = jnp.ones((8, HIDDEN), dtype=jnp.float32)
    s = jax.lax.dot_general(
        ones, prod,
        dimension_numbers=(((1,), (1,)), ((), ())),
        preferred_element_type=jnp.float32)
    out_ref[0, 0, :] = jax.nn.sigmoid(s[0, :])


def kernel(z_i, z_j, R, D, se_indices):
    se3 = se_indices.astype(jnp.int32).reshape(NB, 1, BLK)
    out = pl.pallas_call(
        _body,
        grid=(NB,),
        in_specs=[
            pl.BlockSpec((1, 1, BLK), lambda i: (i, 0, 0)),
            pl.BlockSpec((BLK, HIDDEN), lambda i: (i, 0)),
            pl.BlockSpec((BLK, HIDDEN), lambda i: (i, 0)),
            pl.BlockSpec((HIDDEN, HIDDEN), lambda i: (0, 0)),
            pl.BlockSpec((NUM_SE, HIDDEN), lambda i: (0, 0)),
        ],
        out_specs=pl.BlockSpec((1, 1, BLK), lambda i: (i, 0, 0)),
        out_shape=jax.ShapeDtypeStruct((NB, 1, BLK), jnp.float32),
    )(se3, z_i, z_j, R, D)
    return out.reshape(B)
